# baseline (speedup 1.0000x reference)
import functools

import jax
import jax.numpy as jnp
from jax import lax
from jax.experimental import pallas as pl
from jax.experimental.pallas import tpu as pltpu


def _build_patches(x0, H, W, Wp, Cin):
    S_wide = H * Wp
    lim = 2 * Wp + S_wide
    taps = []
    for kx in range(3):
        xs = x0[kx:kx + lim]
        for ky in range(3):
            taps.append(xs[ky * Wp:ky * Wp + S_wide])
    v = jnp.concatenate(taps, axis=1)
    return v.reshape(H, Wp, 9 * Cin)[:, :W, :].reshape(H * W, 9 * Cin)


def _conv_stats_kernel(x_ref, wt_ref, y_ref, stats_ref, *, H, W, Wp, Cin,
                       Cout):
    p = _build_patches(x_ref[0].astype(jnp.bfloat16), H, W, Wp, Cin)
    y = lax.dot_general(
        wt_ref[...], p, (((1,), (1,)), ((), ())),
        preferred_element_type=jnp.float32)
    y_ref[0] = y.astype(jnp.bfloat16)
    s = jnp.sum(y, axis=1, keepdims=True)
    q = jnp.sum(y * y, axis=1, keepdims=True)
    stats_ref[0] = jnp.concatenate(
        [s, q, jnp.zeros((Cout, 6), jnp.float32)], axis=1)


def _bn_gelu_kernel(y_ref, stats_ref, gamma_ref, beta_ref, o_ref,
                    *, Cout, count, eps, ipb):
    st = jnp.sum(stats_ref[...], axis=0)
    mean = st[:, 0:1] * (1.0 / count)
    var = jnp.maximum(st[:, 1:2] * (1.0 / count) - mean * mean, 0.0)
    scale = gamma_ref[...] * lax.rsqrt(var + eps)
    shift = beta_ref[...] - mean * scale
    for i in range(ipb):
        z = y_ref[i].astype(jnp.float32) * scale + shift
        o_ref[i] = 0.5 * z * (1.0 + lax.erf(z * 0.7071067811865476))


def kernel(x, weight, bias, gamma, beta):
    del bias
    eps = 1e-3
    N, Cin, H, W = x.shape
    Cout = weight.shape[0]
    Wp = (W + 2 + 7) // 8 * 8
    S = H * W
    K = 9 * Cin
    bf16 = jnp.bfloat16
    ipb = 2 if N % 2 == 0 else 1
    FR = (H + 3) * Wp

    x_nhwc = jnp.transpose(x, (0, 2, 3, 1))
    xp = jnp.pad(x_nhwc, ((0, 0), (1, 2), (1, Wp - W - 1), (0, 0)))
    x_flat = xp.reshape(N, FR, Cin)

    wt = jnp.transpose(weight, (0, 3, 2, 1)).reshape(Cout, K).astype(bf16)

    gamma_col = gamma.astype(jnp.float32).reshape(Cout, 1)
    beta_col = beta.astype(jnp.float32).reshape(Cout, 1)

    cparams = pltpu.CompilerParams(
        dimension_semantics=("parallel",),
        vmem_limit_bytes=64 * 1024 * 1024,
    )

    y_bf16, stats = pl.pallas_call(
        functools.partial(_conv_stats_kernel, H=H, W=W, Wp=Wp, Cin=Cin,
                          Cout=Cout),
        grid=(N,),
        in_specs=[
            pl.BlockSpec((1, FR, Cin), lambda n: (n, 0, 0)),
            pl.BlockSpec((Cout, K), lambda n: (0, 0)),
        ],
        out_specs=(
            pl.BlockSpec((1, Cout, S), lambda n: (n, 0, 0)),
            pl.BlockSpec((1, Cout, 8), lambda n: (n, 0, 0)),
        ),
        out_shape=(
            jax.ShapeDtypeStruct((N, Cout, S), bf16),
            jax.ShapeDtypeStruct((N, Cout, 8), jnp.float32),
        ),
        compiler_params=cparams,
    )(x_flat, wt)

    out = pl.pallas_call(
        functools.partial(_bn_gelu_kernel, Cout=Cout,
                          count=float(N * H * W), eps=eps, ipb=ipb),
        grid=(N // ipb,),
        in_specs=[
            pl.BlockSpec((ipb, Cout, S), lambda n: (n, 0, 0)),
            pl.BlockSpec((N, Cout, 8), lambda n: (0, 0, 0)),
            pl.BlockSpec((Cout, 1), lambda n: (0, 0)),
            pl.BlockSpec((Cout, 1), lambda n: (0, 0)),
        ],
        out_specs=pl.BlockSpec((ipb, Cout, S), lambda n: (n, 0, 0)),
        out_shape=jax.ShapeDtypeStruct((N, Cout, S), jnp.float32),
        compiler_params=cparams,
    )(y_bf16, stats, gamma_col, beta_col)

    return out.reshape(N, Cout, H, W)


# speedup vs baseline: 1.0201x; 1.0201x over previous
import functools

import jax
import jax.numpy as jnp
from jax import lax
from jax.experimental import pallas as pl
from jax.experimental.pallas import tpu as pltpu


def _build_patches(x0, H, W, Wp, Cin):
    S_wide = H * Wp
    lim = 2 * Wp + S_wide
    taps = []
    for kx in range(3):
        xs = x0[kx:kx + lim]
        for ky in range(3):
            taps.append(xs[ky * Wp:ky * Wp + S_wide])
    v = jnp.concatenate(taps, axis=1)
    return v.reshape(H, Wp, 9 * Cin)[:, :W, :].reshape(H * W, 9 * Cin)


def _conv_stats_kernel(x_ref, wt_ref, y_ref, stats_ref, *, H, W, Wp, Cin,
                       Cout):
    p = _build_patches(x_ref[0], H, W, Wp, Cin)
    y = lax.dot_general(
        wt_ref[...], p, (((1,), (1,)), ((), ())),
        preferred_element_type=jnp.float32)
    y_ref[0] = y.astype(jnp.bfloat16)
    s = jnp.sum(y, axis=1, keepdims=True)
    q = jnp.sum(y * y, axis=1, keepdims=True)
    stats_ref[0] = jnp.concatenate(
        [s, q, jnp.zeros((Cout, 6), jnp.float32)], axis=1)


def _bn_gelu_kernel(y_ref, stats_ref, gamma_ref, beta_ref, o_ref,
                    *, Cout, count, eps, ipb):
    st = jnp.sum(stats_ref[...], axis=0)
    mean = st[:, 0:1] * (1.0 / count)
    var = jnp.maximum(st[:, 1:2] * (1.0 / count) - mean * mean, 0.0)
    scale = gamma_ref[...] * lax.rsqrt(var + eps)
    shift = beta_ref[...] - mean * scale
    for i in range(ipb):
        z = y_ref[i].astype(jnp.float32) * scale + shift
        o_ref[i] = 0.5 * z * (1.0 + lax.erf(z * 0.7071067811865476))


def kernel(x, weight, bias, gamma, beta):
    del bias
    eps = 1e-3
    N, Cin, H, W = x.shape
    Cout = weight.shape[0]
    Wp = (W + 2 + 7) // 8 * 8
    S = H * W
    K = 9 * Cin
    bf16 = jnp.bfloat16
    ipb = 2 if N % 2 == 0 else 1
    FR = (H + 3) * Wp

    x_nhwc = jnp.transpose(x, (0, 2, 3, 1))
    xp = jnp.pad(x_nhwc, ((0, 0), (1, 2), (1, Wp - W - 1), (0, 0)))
    x_flat = xp.reshape(N, FR, Cin).astype(bf16)

    wt = jnp.transpose(weight, (0, 3, 2, 1)).reshape(Cout, K).astype(bf16)

    gamma_col = gamma.astype(jnp.float32).reshape(Cout, 1)
    beta_col = beta.astype(jnp.float32).reshape(Cout, 1)

    cparams = pltpu.CompilerParams(
        dimension_semantics=("parallel",),
        vmem_limit_bytes=64 * 1024 * 1024,
    )

    y_bf16, stats = pl.pallas_call(
        functools.partial(_conv_stats_kernel, H=H, W=W, Wp=Wp, Cin=Cin,
                          Cout=Cout),
        grid=(N,),
        in_specs=[
            pl.BlockSpec((1, FR, Cin), lambda n: (n, 0, 0)),
            pl.BlockSpec((Cout, K), lambda n: (0, 0)),
        ],
        out_specs=(
            pl.BlockSpec((1, Cout, S), lambda n: (n, 0, 0)),
            pl.BlockSpec((1, Cout, 8), lambda n: (n, 0, 0)),
        ),
        out_shape=(
            jax.ShapeDtypeStruct((N, Cout, S), bf16),
            jax.ShapeDtypeStruct((N, Cout, 8), jnp.float32),
        ),
        compiler_params=cparams,
    )(x_flat, wt)

    out = pl.pallas_call(
        functools.partial(_bn_gelu_kernel, Cout=Cout,
                          count=float(N * H * W), eps=eps, ipb=ipb),
        grid=(N // ipb,),
        in_specs=[
            pl.BlockSpec((ipb, Cout, S), lambda n: (n, 0, 0)),
            pl.BlockSpec((N, Cout, 8), lambda n: (0, 0, 0)),
            pl.BlockSpec((Cout, 1), lambda n: (0, 0)),
            pl.BlockSpec((Cout, 1), lambda n: (0, 0)),
        ],
        out_specs=pl.BlockSpec((ipb, Cout, S), lambda n: (n, 0, 0)),
        out_shape=jax.ShapeDtypeStruct((N, Cout, S), jnp.float32),
        compiler_params=cparams,
    )(y_bf16, stats, gamma_col, beta_col)

    return out.reshape(N, Cout, H, W)
